# baseline (speedup 1.0000x reference)
import functools

import jax
import jax.numpy as jnp
import numpy as np
from jax.experimental import pallas as pl
from jax.experimental.pallas import tpu as pltpu
from jax.experimental.shard_map import shard_map
from jax.sharding import Mesh, PartitionSpec as P


def _leaky(x, slope=0.1):
    return jnp.where(x >= 0, x, slope * x)


def _gnn_kernel(NB, C, E, NI, H,
                x_ref, eidx_ref, etype_ref,
                w1_ref, b1_ref, w2_ref, b2_ref,
                wf_ref, wt_ref, wet_ref, bed_ref,
                wsec_ref, bsec_ref, out_ref,
                cur_ref, oh_ref, scat_ref):
    f32 = jnp.float32
    bf16 = jnp.bfloat16
    j = pl.program_id(0)

    @pl.when(j == 0)
    def _encoder_stage():
        x = x_ref[...]
        h = jnp.dot(x, w1_ref[...], preferred_element_type=f32) + b1_ref[...]
        h = _leaky(_leaky(h))
        h = _leaky(jnp.dot(h.astype(bf16), w2_ref[...].astype(bf16),
                           preferred_element_type=f32) + b2_ref[...])
        cur_ref[...] = h
        out_ref[...] = jnp.dot(h.astype(bf16), wsec_ref[0].astype(bf16),
                               preferred_element_type=f32)

        lane = jax.lax.broadcasted_iota(jnp.int32, (2 * E, C), 1)
        for b in range(NB):
            eb = eidx_ref[b]
            gft = jnp.concatenate([eb[:, 0:1], eb[:, 1:2]], axis=0)
            oh = (lane == gft).astype(f32)
            oh_ref[b] = oh
            scat_ref[b] = oh[:E, :].T

    @pl.when(j > 0)
    def _iter_stage():
        cur = cur_ref[...].astype(bf16)
        gs = [jnp.dot(oh_ref[b].astype(bf16), cur[b * C:(b + 1) * C, :],
                      preferred_element_type=f32) for b in range(NB)]
        gf = jnp.concatenate([g[:E, :] for g in gs], axis=0)
        gt = jnp.concatenate([g[E:, :] for g in gs], axis=0)
        z = (jnp.dot(gf.astype(bf16), wf_ref[0].astype(bf16),
                     preferred_element_type=f32)
             + jnp.dot(gt.astype(bf16), wt_ref[0].astype(bf16),
                       preferred_element_type=f32)
             + jnp.dot(etype_ref[...], wet_ref[0], preferred_element_type=f32))
        z = jnp.maximum(z + bed_ref[0], 0.0).astype(bf16)
        new_cur = jnp.concatenate(
            [jnp.dot(scat_ref[b].astype(bf16), z[b * E:(b + 1) * E, :],
                     preferred_element_type=f32) for b in range(NB)], axis=0)
        cur_ref[...] = new_cur
        out_ref[...] = out_ref[...] + jnp.dot(
            new_cur.astype(bf16), wsec_ref[0].astype(bf16),
            preferred_element_type=f32)

    @pl.when(j == NI)
    def _finalize_stage():
        out_ref[...] = _leaky(out_ref[...] + bsec_ref[...])


def _forward_local(child_feats, edge_indices, edge_type_onehot,
                   w1, b1, w2, b2, wf, wt, wet, bed, wsec, bsec):
    NB, C, Fin = child_feats.shape
    E = edge_indices.shape[1]
    T = edge_type_onehot.shape[2]
    NI, _, H = wf.shape
    F_out = wsec.shape[2]
    f32 = jnp.float32

    x = child_feats.astype(f32).reshape(NB * C, Fin)
    eidx = edge_indices.astype(jnp.int32)
    etype = edge_type_onehot.astype(f32).reshape(NB * E, T)

    def slab(j):
        return jnp.maximum(j - 1, 0)

    kern = functools.partial(_gnn_kernel, NB, C, E, NI, H)
    out = pl.pallas_call(
        kern,
        out_shape=jax.ShapeDtypeStruct((NB * C, F_out), f32),
        grid=(NI + 1,),
        in_specs=[
            pl.BlockSpec((NB * C, Fin), lambda j: (0, 0)),
            pl.BlockSpec((NB, E, 2), lambda j: (0, 0, 0)),
            pl.BlockSpec((NB * E, T), lambda j: (0, 0)),
            pl.BlockSpec((Fin, H), lambda j: (0, 0)),
            pl.BlockSpec((1, H), lambda j: (0, 0)),
            pl.BlockSpec((H, H), lambda j: (0, 0)),
            pl.BlockSpec((1, H), lambda j: (0, 0)),
            pl.BlockSpec((1, H, H), lambda j: (slab(j), 0, 0)),
            pl.BlockSpec((1, H, H), lambda j: (slab(j), 0, 0)),
            pl.BlockSpec((1, T, H), lambda j: (slab(j), 0, 0)),
            pl.BlockSpec((1, 1, H), lambda j: (slab(j), 0, 0)),
            pl.BlockSpec((1, H, F_out), lambda j: (j, 0, 0)),
            pl.BlockSpec((1, F_out), lambda j: (0, 0)),
        ],
        out_specs=pl.BlockSpec((NB * C, F_out), lambda j: (0, 0)),
        scratch_shapes=[
            pltpu.VMEM((NB * C, H), f32),
            pltpu.VMEM((NB, 2 * E, C), f32),
            pltpu.VMEM((NB, C, E), f32),
        ],
        compiler_params=pltpu.CompilerParams(
            dimension_semantics=("arbitrary",)),
    )(x, eidx, etype, w1, b1, w2, b2, wf, wt, wet, bed, wsec, bsec)
    return out.reshape(NB, C, F_out)


def kernel(child_feats, edge_indices, edge_type_onehot, lengths,
           w1, b1, w2, b2, wf, wt, wet, bed, wsec, bsec):
    del lengths
    B = child_feats.shape[0]
    devs = jax.devices()
    ndev = 2 if (len(devs) >= 2 and B % 2 == 0) else 1
    mesh = Mesh(np.array(devs[:ndev]), ("x",))
    shard = P("x")
    repl = P()
    fwd = shard_map(
        _forward_local, mesh=mesh,
        in_specs=(shard, shard, shard,
                  repl, repl, repl, repl, repl, repl, repl, repl, repl, repl),
        out_specs=shard, check_rep=False,
    )
    return fwd(child_feats, edge_indices, edge_type_onehot,
               w1, b1, w2, b2, wf, wt, wet, bed, wsec, bsec)


# speedup vs baseline: 8.7452x; 8.7452x over previous
import functools

import jax
import jax.numpy as jnp
from jax.experimental import pallas as pl
from jax.experimental.pallas import tpu as pltpu


def _leaky(x, slope=0.1):
    return jnp.where(x >= 0, x, slope * x)


def _gnn_kernel(NB, C, E, NI, H,
                x_ref, eidx_ref, etype_ref,
                w1_ref, b1_ref, w2_ref, b2_ref,
                wf_ref, wt_ref, wet_ref, bed_ref,
                wsec_ref, bsec_ref, out_ref,
                cur_ref, oh_ref, scat_ref):
    f32 = jnp.float32
    bf16 = jnp.bfloat16
    j = pl.program_id(0)

    @pl.when(j == 0)
    def _encoder_stage():
        x = x_ref[...]
        h = jnp.dot(x, w1_ref[...], preferred_element_type=f32) + b1_ref[...]
        h = _leaky(_leaky(h))
        h = _leaky(jnp.dot(h.astype(bf16), w2_ref[...].astype(bf16),
                           preferred_element_type=f32) + b2_ref[...])
        cur_ref[...] = h
        out_ref[...] = jnp.dot(h.astype(bf16), wsec_ref[0].astype(bf16),
                               preferred_element_type=f32)

        lane = jax.lax.broadcasted_iota(jnp.int32, (2 * E, C), 1)
        for b in range(NB):
            eb = eidx_ref[b]
            gft = jnp.concatenate([eb[:, 0:1], eb[:, 1:2]], axis=0)
            oh = (lane == gft).astype(f32)
            oh_ref[b] = oh
            scat_ref[b] = oh[:E, :].T

    @pl.when(j > 0)
    def _iter_stage():
        cur = cur_ref[...].astype(bf16)
        gs = [jnp.dot(oh_ref[b].astype(bf16), cur[b * C:(b + 1) * C, :],
                      preferred_element_type=f32) for b in range(NB)]
        gf = jnp.concatenate([g[:E, :] for g in gs], axis=0)
        gt = jnp.concatenate([g[E:, :] for g in gs], axis=0)
        z = (jnp.dot(gf.astype(bf16), wf_ref[0].astype(bf16),
                     preferred_element_type=f32)
             + jnp.dot(gt.astype(bf16), wt_ref[0].astype(bf16),
                       preferred_element_type=f32)
             + jnp.dot(etype_ref[...], wet_ref[0], preferred_element_type=f32))
        z = jnp.maximum(z + bed_ref[0], 0.0).astype(bf16)
        new_cur = jnp.concatenate(
            [jnp.dot(scat_ref[b].astype(bf16), z[b * E:(b + 1) * E, :],
                     preferred_element_type=f32) for b in range(NB)], axis=0)
        cur_ref[...] = new_cur
        out_ref[...] = out_ref[...] + jnp.dot(
            new_cur.astype(bf16), wsec_ref[0].astype(bf16),
            preferred_element_type=f32)

    @pl.when(j == NI)
    def _finalize_stage():
        out_ref[...] = _leaky(out_ref[...] + bsec_ref[...])


def _forward_local(child_feats, edge_indices, edge_type_onehot,
                   w1, b1, w2, b2, wf, wt, wet, bed, wsec, bsec):
    NB, C, Fin = child_feats.shape
    E = edge_indices.shape[1]
    T = edge_type_onehot.shape[2]
    NI, _, H = wf.shape
    F_out = wsec.shape[2]
    f32 = jnp.float32

    x = child_feats.astype(f32).reshape(NB * C, Fin)
    eidx = edge_indices.astype(jnp.int32)
    etype = edge_type_onehot.astype(f32).reshape(NB * E, T)

    def slab(j):
        return jnp.maximum(j - 1, 0)

    kern = functools.partial(_gnn_kernel, NB, C, E, NI, H)
    out = pl.pallas_call(
        kern,
        out_shape=jax.ShapeDtypeStruct((NB * C, F_out), f32),
        grid=(NI + 1,),
        in_specs=[
            pl.BlockSpec((NB * C, Fin), lambda j: (0, 0)),
            pl.BlockSpec((NB, E, 2), lambda j: (0, 0, 0)),
            pl.BlockSpec((NB * E, T), lambda j: (0, 0)),
            pl.BlockSpec((Fin, H), lambda j: (0, 0)),
            pl.BlockSpec((1, H), lambda j: (0, 0)),
            pl.BlockSpec((H, H), lambda j: (0, 0)),
            pl.BlockSpec((1, H), lambda j: (0, 0)),
            pl.BlockSpec((1, H, H), lambda j: (slab(j), 0, 0)),
            pl.BlockSpec((1, H, H), lambda j: (slab(j), 0, 0)),
            pl.BlockSpec((1, T, H), lambda j: (slab(j), 0, 0)),
            pl.BlockSpec((1, 1, H), lambda j: (slab(j), 0, 0)),
            pl.BlockSpec((1, H, F_out), lambda j: (j, 0, 0)),
            pl.BlockSpec((1, F_out), lambda j: (0, 0)),
        ],
        out_specs=pl.BlockSpec((NB * C, F_out), lambda j: (0, 0)),
        scratch_shapes=[
            pltpu.VMEM((NB * C, H), f32),
            pltpu.VMEM((NB, 2 * E, C), f32),
            pltpu.VMEM((NB, C, E), f32),
        ],
        compiler_params=pltpu.CompilerParams(
            dimension_semantics=("arbitrary",)),
    )(x, eidx, etype, w1, b1, w2, b2, wf, wt, wet, bed, wsec, bsec)
    return out.reshape(NB, C, F_out)


@jax.jit
def kernel(child_feats, edge_indices, edge_type_onehot, lengths,
           w1, b1, w2, b2, wf, wt, wet, bed, wsec, bsec):
    del lengths
    return _forward_local(child_feats, edge_indices, edge_type_onehot,
                          w1, b1, w2, b2, wf, wt, wet, bed, wsec, bsec)
